# baseline (speedup 1.0000x reference)
import jax
import jax.numpy as jnp
from jax.experimental import pallas as pl

B, T, C = 256, 32, 128
NH, HD = 4, 32
NE, TOPK, DFF = 8, 2, 512

BB = 64
P = BB * T


def _fused_kernel(x_ref, epsT_ref, wq_ref, wk_ref, wv_ref, wp_ref, bp_ref,
                  ln1g_ref, ln1b_ref, ln2g_ref, ln2b_ref,
                  wrt_ref, brt_ref, we1_ref, be1_ref, we2_ref, be2_ref,
                  selcat_ref, out_ref):
    f32 = jnp.float32
    bf = jnp.bfloat16
    xb = x_ref[...]
    x2 = xb.reshape(P, C)

    m = jnp.mean(x2, axis=-1, keepdims=True)
    ms = jnp.mean(x2 * x2, axis=-1, keepdims=True)
    r = jax.lax.rsqrt(ms - m * m + 1e-5)
    xn = (x2 - m) * r * ln1g_ref[...] + ln1b_ref[...]

    xnb = xn.astype(bf)
    q = jnp.dot(xnb, wq_ref[...], preferred_element_type=f32).astype(bf)
    k = jnp.dot(xnb, wk_ref[...], preferred_element_type=f32).astype(bf)
    vv = jnp.dot(xnb, wv_ref[...], preferred_element_type=f32).astype(bf)

    row = jax.lax.broadcasted_iota(jnp.int32, (BB, T, T), 1)
    col = jax.lax.broadcasted_iota(jnp.int32, (BB, T, T), 2)
    neg_mask = jnp.where(row >= col, 0.0, -1e30).astype(bf)

    ones_col = jnp.ones((BB, T, 1), bf)
    att_cols = []
    for h in range(NH):
        qh = q[:, h * HD:(h + 1) * HD].reshape(BB, T, HD)
        kh = k[:, h * HD:(h + 1) * HD].reshape(BB, T, HD)
        vh = vv[:, h * HD:(h + 1) * HD].reshape(BB, T, HD)
        s = jax.lax.dot_general(
            qh, kh, (((2,), (2,)), ((0,), (0,))),
            preferred_element_type=f32)
        e = jnp.exp(s.astype(bf) + neg_mask)
        vh_aug = jnp.concatenate([vh, ones_col], axis=-1)
        ah_aug = jax.lax.dot_general(
            e, vh_aug, (((2,), (1,)), ((0,), (0,))),
            preferred_element_type=f32)
        ah = ah_aug[:, :, :HD] * jax.lax.reciprocal(ah_aug[:, :, HD:])
        att_cols.append(ah.astype(bf).reshape(P, HD))
    att = jnp.concatenate(att_cols, axis=-1)

    x1 = x2 + jnp.dot(att, wp_ref[...],
                      preferred_element_type=f32) + bp_ref[...]

    m2 = jnp.mean(x1, axis=-1, keepdims=True)
    ms2 = jnp.mean(x1 * x1, axis=-1, keepdims=True)
    r2 = jax.lax.rsqrt(ms2 - m2 * m2 + 1e-5)
    xn2 = (x1 - m2) * r2 * ln2g_ref[...] + ln2b_ref[...]

    sT = jax.lax.dot_general(
        wrt_ref[...], xn2, (((1,), (1,)), ((), ())),
        preferred_element_type=f32) + brt_ref[...]
    logitsT = sT[:NE]
    nlogT = sT[NE:2 * NE]
    skipT = sT[2 * NE:2 * NE + 1]
    noisyT = logitsT + epsT_ref[...] * jax.nn.softplus(nlogT)

    idxs = jax.lax.broadcasted_iota(jnp.int32, (NE, P), 0)
    v1 = jnp.max(noisyT, axis=0, keepdims=True)
    i1 = jnp.min(jnp.where(noisyT == v1, idxs, NE), axis=0, keepdims=True)
    n2 = jnp.where(idxs == i1, -jnp.inf, noisyT)
    v2 = jnp.max(n2, axis=0, keepdims=True)
    i2 = jnp.min(jnp.where(n2 == v2, idxs, NE), axis=0, keepdims=True)
    sel = (idxs == i1) | (idxs == i2)
    prT = jnp.where(sel, jnp.exp(noisyT - v1), 0.0)
    prT = prT / jnp.sum(prT, axis=0, keepdims=True)

    skip_col = jnp.transpose(skipT)
    pr_col = jnp.transpose(prT)

    f8 = jnp.float8_e4m3fn
    xn2q = xn2.astype(f8)
    acc = jnp.zeros((P, C), f32)
    for e in range(NE):
        h = jnp.dot(xn2q, we1_ref[e], preferred_element_type=f32)
        hb = jnp.maximum(h + be1_ref[e], 0).astype(f8)
        ye = jnp.dot(hb, we2_ref[e], preferred_element_type=f32)
        acc = acc + ye * pr_col[:, e:e + 1]
    acc = acc * (1.0 / 256.0) \
        + jnp.dot(pr_col, be2_ref[...], preferred_element_type=f32)

    out = jnp.where(skip_col > 0.0, x1, x1 + acc)
    out_ref[...] = out.reshape(BB, T, C)


def kernel(x, Wq, Wk, Wv, Wp, bp, ln1_g, ln1_b, ln2_g, ln2_b,
           Wer, ber, Wn, bn, Ws, bs, We1, be1, We2, be2):
    bf = jnp.bfloat16
    Wqf = (Wq.transpose(1, 0, 2).reshape(C, NH * HD) * C ** -0.5).astype(bf)
    Wkf = Wk.transpose(1, 0, 2).reshape(C, NH * HD).astype(bf)
    Wvf = Wv.transpose(1, 0, 2).reshape(C, NH * HD).astype(bf)
    Wpb = Wp.astype(bf)
    f8 = jnp.float8_e4m3fn
    We1b = (We1 * 16.0).astype(f8)
    We2b = (We2 * 16.0).astype(f8)
    be1b = be1 * 16.0
    selcat = jnp.repeat(jnp.eye(NE, dtype=bf), DFF, axis=1)
    WrT = jnp.concatenate([Wer, Wn, Ws], axis=1).T
    brT = jnp.concatenate([ber, bn, bs]).reshape(2 * NE + 1, 1)
    eps = jax.random.normal(jax.random.key(42), (B, T, NE), jnp.float32)
    epsT = eps.reshape(B * T, NE).T

    row = lambda a: a.reshape(1, -1)
    full = lambda arr: pl.BlockSpec(arr.shape, lambda i: (0,) * arr.ndim)

    weights = (Wqf, Wkf, Wvf, Wpb, row(bp), row(ln1_g), row(ln1_b),
               row(ln2_g), row(ln2_b), WrT, brT, We1b, be1b, We2b, be2,
               selcat)

    out = pl.pallas_call(
        _fused_kernel,
        grid=(B // BB,),
        in_specs=[pl.BlockSpec((BB, T, C), lambda i: (i, 0, 0)),
                  pl.BlockSpec((NE, P), lambda i: (0, i))]
                 + [full(w) for w in weights],
        out_specs=pl.BlockSpec((BB, T, C), lambda i: (i, 0, 0)),
        out_shape=jax.ShapeDtypeStruct((B, T, C), jnp.float32),
    )(x, epsT, *weights)
    return out


# speedup vs baseline: 1.0037x; 1.0037x over previous
import jax
import jax.numpy as jnp
from jax.experimental import pallas as pl

B, T, C = 256, 32, 128
NH, HD = 4, 32
NE, TOPK, DFF = 8, 2, 512

BB = 32
P = BB * T


def _fused_kernel(x_ref, epsT_ref, wq_ref, wk_ref, wv_ref, wp_ref, bp_ref,
                  ln1g_ref, ln1b_ref, ln2g_ref, ln2b_ref,
                  wrt_ref, brt_ref, we1_ref, be1_ref, we2_ref, be2_ref,
                  selcat_ref, out_ref):
    f32 = jnp.float32
    bf = jnp.bfloat16
    xb = x_ref[...]
    x2 = xb.reshape(P, C)

    m = jnp.mean(x2, axis=-1, keepdims=True)
    ms = jnp.mean(x2 * x2, axis=-1, keepdims=True)
    r = jax.lax.rsqrt(ms - m * m + 1e-5)

    xnb = ((x2 - m) * r).astype(bf) * ln1g_ref[...] + ln1b_ref[...]
    q = jnp.dot(xnb, wq_ref[...], preferred_element_type=f32).astype(bf)
    k = jnp.dot(xnb, wk_ref[...], preferred_element_type=f32).astype(bf)
    vv = jnp.dot(xnb, wv_ref[...], preferred_element_type=f32).astype(bf)

    row = jax.lax.broadcasted_iota(jnp.int32, (BB, T, T), 1)
    col = jax.lax.broadcasted_iota(jnp.int32, (BB, T, T), 2)
    neg_mask = jnp.where(row >= col, 0.0, -1e30).astype(bf)

    ones_col = jnp.ones((BB, T, 1), bf)
    att_cols = []
    for h in range(NH):
        qh = q[:, h * HD:(h + 1) * HD].reshape(BB, T, HD)
        kh = k[:, h * HD:(h + 1) * HD].reshape(BB, T, HD)
        vh = vv[:, h * HD:(h + 1) * HD].reshape(BB, T, HD)
        s = jax.lax.dot_general(
            qh, kh, (((2,), (2,)), ((0,), (0,))),
            preferred_element_type=f32)
        e = jnp.exp(s.astype(bf) + neg_mask)
        vh_aug = jnp.concatenate([vh, ones_col], axis=-1)
        ah_aug = jax.lax.dot_general(
            e, vh_aug, (((2,), (1,)), ((0,), (0,))),
            preferred_element_type=f32)
        ah = ah_aug[:, :, :HD] * jax.lax.reciprocal(ah_aug[:, :, HD:])
        att_cols.append(ah.astype(bf).reshape(P, HD))
    att = jnp.concatenate(att_cols, axis=-1)

    x1 = x2 + jnp.dot(att, wp_ref[...],
                      preferred_element_type=f32) + bp_ref[...]

    m2 = jnp.mean(x1, axis=-1, keepdims=True)
    ms2 = jnp.mean(x1 * x1, axis=-1, keepdims=True)
    r2 = jax.lax.rsqrt(ms2 - m2 * m2 + 1e-5)
    xn2 = (x1 - m2) * r2 * ln2g_ref[...] + ln2b_ref[...]

    sT = jax.lax.dot_general(
        wrt_ref[...], xn2, (((1,), (1,)), ((), ())),
        preferred_element_type=f32) + brt_ref[...]
    logitsT = sT[:NE]
    nlogT = sT[NE:2 * NE]
    skipT = sT[2 * NE:2 * NE + 1]
    noisyT = logitsT + epsT_ref[...] * jax.nn.softplus(nlogT)

    idxs = jax.lax.broadcasted_iota(jnp.int32, (NE, P), 0)
    v1 = jnp.max(noisyT, axis=0, keepdims=True)
    i1 = jnp.min(jnp.where(noisyT == v1, idxs, NE), axis=0, keepdims=True)
    n2 = jnp.where(idxs == i1, -jnp.inf, noisyT)
    v2 = jnp.max(n2, axis=0, keepdims=True)
    i2 = jnp.min(jnp.where(n2 == v2, idxs, NE), axis=0, keepdims=True)
    sel = (idxs == i1) | (idxs == i2)
    prT = jnp.where(sel, jnp.exp(noisyT - v1), 0.0)
    prT = prT / jnp.sum(prT, axis=0, keepdims=True)

    skip_col = jnp.transpose(skipT)
    pr_col = jnp.transpose(prT)

    f8 = jnp.float8_e4m3fn
    xn2q = xn2.astype(f8)
    acc = jnp.zeros((P, C), f32)
    for e in range(NE):
        h = jnp.dot(xn2q, we1_ref[e], preferred_element_type=f32)
        hb = jnp.maximum(h + be1_ref[e], 0).astype(f8)
        ye = jnp.dot(hb, we2_ref[e], preferred_element_type=f32)
        acc = acc + ye * pr_col[:, e:e + 1]
    acc = acc * (1.0 / 256.0) \
        + jnp.dot(pr_col, be2_ref[...], preferred_element_type=f32)

    out = jnp.where(skip_col > 0.0, x1, x1 + acc)
    out_ref[...] = out.reshape(BB, T, C)


def kernel(x, Wq, Wk, Wv, Wp, bp, ln1_g, ln1_b, ln2_g, ln2_b,
           Wer, ber, Wn, bn, Ws, bs, We1, be1, We2, be2):
    bf = jnp.bfloat16
    Wqf = (Wq.transpose(1, 0, 2).reshape(C, NH * HD) * C ** -0.5).astype(bf)
    Wkf = Wk.transpose(1, 0, 2).reshape(C, NH * HD).astype(bf)
    Wvf = Wv.transpose(1, 0, 2).reshape(C, NH * HD).astype(bf)
    Wpb = Wp.astype(bf)
    f8 = jnp.float8_e4m3fn
    We1b = (We1 * 16.0).astype(f8)
    We2b = (We2 * 16.0).astype(f8)
    be1b = be1 * 16.0
    selcat = jnp.repeat(jnp.eye(NE, dtype=bf), DFF, axis=1)
    WrT = jnp.concatenate([Wer, Wn, Ws], axis=1).T
    brT = jnp.concatenate([ber, bn, bs]).reshape(2 * NE + 1, 1)
    eps = jax.random.normal(jax.random.key(42), (B, T, NE), jnp.float32)
    epsT = eps.reshape(B * T, NE).T

    row = lambda a: a.reshape(1, -1)
    full = lambda arr: pl.BlockSpec(arr.shape, lambda i: (0,) * arr.ndim)

    weights = (Wqf, Wkf, Wvf, Wpb, row(bp),
               row(ln1_g).astype(bf), row(ln1_b).astype(bf),
               row(ln2_g), row(ln2_b), WrT, brT, We1b, be1b, We2b, be2,
               selcat)

    out = pl.pallas_call(
        _fused_kernel,
        grid=(B // BB,),
        in_specs=[pl.BlockSpec((BB, T, C), lambda i: (i, 0, 0)),
                  pl.BlockSpec((NE, P), lambda i: (0, i))]
                 + [full(w) for w in weights],
        out_specs=pl.BlockSpec((BB, T, C), lambda i: (i, 0, 0)),
        out_shape=jax.ShapeDtypeStruct((B, T, C), jnp.float32),
    )(x, epsT, *weights)
    return out


# speedup vs baseline: 1.0213x; 1.0175x over previous
import jax
import jax.numpy as jnp
from jax.experimental import pallas as pl

B, T, C = 256, 32, 128
NH, HD = 4, 32
NE, TOPK, DFF = 8, 2, 512

BB = 32
P = BB * T


def _fused_kernel(x_ref, epsT_ref, wq_ref, wk_ref, wv_ref, wp_ref, bp_ref,
                  ln1g_ref, ln1b_ref, ln2g_ref, ln2b_ref,
                  wrt_ref, brt_ref, we1_ref, be1_ref, we2_ref, be2_ref,
                  selcat_ref, out_ref):
    f32 = jnp.float32
    bf = jnp.bfloat16
    xb = x_ref[...]
    x2 = xb.reshape(P, C)

    m = jnp.mean(x2, axis=-1, keepdims=True)
    ms = jnp.mean(x2 * x2, axis=-1, keepdims=True)
    r = jax.lax.rsqrt(ms - m * m + 1e-5)
    xn = (x2 - m) * r * ln1g_ref[...] + ln1b_ref[...]

    xnb = xn.astype(bf)
    q = jnp.dot(xnb, wq_ref[...], preferred_element_type=f32).astype(bf)
    k = jnp.dot(xnb, wk_ref[...], preferred_element_type=f32).astype(bf)
    vv = jnp.dot(xnb, wv_ref[...], preferred_element_type=f32).astype(bf)

    row = jax.lax.broadcasted_iota(jnp.int32, (BB, T, T), 1)
    col = jax.lax.broadcasted_iota(jnp.int32, (BB, T, T), 2)
    neg_mask = jnp.where(row >= col, 0.0, -1e30).astype(bf)

    ones_col = jnp.ones((BB, T, 1), bf)
    att_cols = []
    for h in range(NH):
        qh = q[:, h * HD:(h + 1) * HD].reshape(BB, T, HD)
        kh = k[:, h * HD:(h + 1) * HD].reshape(BB, T, HD)
        vh = vv[:, h * HD:(h + 1) * HD].reshape(BB, T, HD)
        s = jax.lax.dot_general(
            qh, kh, (((2,), (2,)), ((0,), (0,))),
            preferred_element_type=f32)
        e = jnp.exp(s.astype(bf) + neg_mask)
        vh_aug = jnp.concatenate([vh, ones_col], axis=-1)
        ah_aug = jax.lax.dot_general(
            e, vh_aug, (((2,), (1,)), ((0,), (0,))),
            preferred_element_type=f32)
        ah = ah_aug[:, :, :HD] * jax.lax.reciprocal(ah_aug[:, :, HD:])
        att_cols.append(ah.astype(bf).reshape(P, HD))
    att = jnp.concatenate(att_cols, axis=-1)

    x1 = x2 + jnp.dot(att, wp_ref[...],
                      preferred_element_type=f32) + bp_ref[...]

    m2 = jnp.mean(x1, axis=-1, keepdims=True)
    ms2 = jnp.mean(x1 * x1, axis=-1, keepdims=True)
    r2 = jax.lax.rsqrt(ms2 - m2 * m2 + 1e-5)
    xn2 = (x1 - m2) * r2 * ln2g_ref[...] + ln2b_ref[...]

    sT = jax.lax.dot_general(
        wrt_ref[...], xn2, (((1,), (1,)), ((), ())),
        preferred_element_type=f32) + brt_ref[...]
    logitsT = sT[:NE]
    nlogT = sT[NE:2 * NE]
    skipT = sT[2 * NE:2 * NE + 1]
    noisyT = logitsT + epsT_ref[...] * jax.nn.softplus(nlogT)

    idxs = jax.lax.broadcasted_iota(jnp.int32, (NE, P), 0)
    v1 = jnp.max(noisyT, axis=0, keepdims=True)
    i1 = jnp.min(jnp.where(noisyT == v1, idxs, NE), axis=0, keepdims=True)
    n2 = jnp.where(idxs == i1, -jnp.inf, noisyT)
    v2 = jnp.max(n2, axis=0, keepdims=True)
    i2 = jnp.min(jnp.where(n2 == v2, idxs, NE), axis=0, keepdims=True)
    sel = (idxs == i1) | (idxs == i2)
    prT = jnp.where(sel, jnp.exp(noisyT - v1), 0.0)
    prT = prT / jnp.sum(prT, axis=0, keepdims=True)

    skip_col = jnp.transpose(skipT)
    pr_col = jnp.transpose(prT)

    f8 = jnp.float8_e4m3fn
    xn2q = xn2.astype(f8)
    acc = jnp.zeros((P, C), f32)
    for e in range(NE):
        h = jnp.dot(xn2q, we1_ref[e], preferred_element_type=f32)
        hb = jnp.maximum(h + be1_ref[e], 0).astype(f8)
        ye = jnp.dot(hb, we2_ref[e], preferred_element_type=f32)
        acc = acc + ye * pr_col[:, e:e + 1]
    acc = acc * (1.0 / 256.0) \
        + jnp.dot(pr_col, be2_ref[...], preferred_element_type=f32)

    out = jnp.where(skip_col > 0.0, x1, x1 + acc)
    out_ref[...] = out.reshape(BB, T, C)


def kernel(x, Wq, Wk, Wv, Wp, bp, ln1_g, ln1_b, ln2_g, ln2_b,
           Wer, ber, Wn, bn, Ws, bs, We1, be1, We2, be2):
    bf = jnp.bfloat16
    Wqf = (Wq.transpose(1, 0, 2).reshape(C, NH * HD) * C ** -0.5).astype(bf)
    Wkf = Wk.transpose(1, 0, 2).reshape(C, NH * HD).astype(bf)
    Wvf = Wv.transpose(1, 0, 2).reshape(C, NH * HD).astype(bf)
    Wpb = Wp.astype(bf)
    f8 = jnp.float8_e4m3fn
    We1b = (We1 * 16.0).astype(f8)
    We2b = (We2 * 16.0).astype(f8)
    be1b = be1 * 16.0
    selcat = jnp.repeat(jnp.eye(NE, dtype=bf), DFF, axis=1)
    WrT = jnp.concatenate([Wer, Wn, Ws], axis=1).T
    brT = jnp.concatenate([ber, bn, bs]).reshape(2 * NE + 1, 1)
    eps = jax.random.normal(jax.random.key(42), (B, T, NE), jnp.float32)
    epsT = eps.reshape(B * T, NE).T

    row = lambda a: a.reshape(1, -1)
    full = lambda arr: pl.BlockSpec(arr.shape, lambda i: (0,) * arr.ndim)

    weights = (Wqf, Wkf, Wvf, Wpb, row(bp), row(ln1_g), row(ln1_b),
               row(ln2_g), row(ln2_b), WrT, brT, We1b, be1b, We2b, be2,
               selcat)

    out = pl.pallas_call(
        _fused_kernel,
        grid=(B // BB,),
        in_specs=[pl.BlockSpec((BB, T, C), lambda i: (i, 0, 0)),
                  pl.BlockSpec((NE, P), lambda i: (0, i))]
                 + [full(w) for w in weights],
        out_specs=pl.BlockSpec((BB, T, C), lambda i: (i, 0, 0)),
        out_shape=jax.ShapeDtypeStruct((B, T, C), jnp.float32),
    )(x, epsT, *weights)
    return out


# speedup vs baseline: 1.0360x; 1.0144x over previous
import jax
import jax.numpy as jnp
from jax.experimental import pallas as pl

B, T, C = 256, 32, 128
NH, HD = 4, 32
NE, TOPK, DFF = 8, 2, 512

BB = 32
P = BB * T


def _fused_kernel(x_ref, epsT_ref, wq_ref, wk_ref, wv_ref, wp_ref, bp_ref,
                  ln1g_ref, ln1b_ref, ln2g_ref, ln2b_ref,
                  wrt_ref, brt_ref, we1_ref, be1_ref, we2_ref, be2_ref,
                  out_ref):
    f32 = jnp.float32
    bf = jnp.bfloat16
    xb = x_ref[...]
    x2 = xb.reshape(P, C)

    m = jnp.mean(x2, axis=-1, keepdims=True)
    ms = jnp.mean(x2 * x2, axis=-1, keepdims=True)
    r = jax.lax.rsqrt(ms - m * m + 1e-5)
    xn = (x2 - m) * r * ln1g_ref[...] + ln1b_ref[...]

    xnb = xn.astype(bf)
    q = jnp.dot(xnb, wq_ref[...], preferred_element_type=f32).astype(bf)
    k = jnp.dot(xnb, wk_ref[...], preferred_element_type=f32).astype(bf)
    vv = jnp.dot(xnb, wv_ref[...], preferred_element_type=f32).astype(bf)

    row = jax.lax.broadcasted_iota(jnp.int32, (BB, T, T), 1)
    col = jax.lax.broadcasted_iota(jnp.int32, (BB, T, T), 2)
    neg_mask = jnp.where(row >= col, 0.0, -1e30).astype(bf)

    ones_col = jnp.ones((BB, T, 1), bf)
    att_cols = []
    for h in range(NH):
        qh = q[:, h * HD:(h + 1) * HD].reshape(BB, T, HD)
        kh = k[:, h * HD:(h + 1) * HD].reshape(BB, T, HD)
        vh = vv[:, h * HD:(h + 1) * HD].reshape(BB, T, HD)
        s = jax.lax.dot_general(
            qh, kh, (((2,), (2,)), ((0,), (0,))),
            preferred_element_type=f32)
        e = jnp.exp(s.astype(bf) + neg_mask)
        vh_aug = jnp.concatenate([vh, ones_col], axis=-1)
        ah_aug = jax.lax.dot_general(
            e, vh_aug, (((2,), (1,)), ((0,), (0,))),
            preferred_element_type=f32)
        ah = ah_aug[:, :, :HD] * jax.lax.reciprocal(ah_aug[:, :, HD:])
        att_cols.append(ah.astype(bf).reshape(P, HD))
    att = jnp.concatenate(att_cols, axis=-1)

    x1 = x2 + jnp.dot(att, wp_ref[...],
                      preferred_element_type=f32) + bp_ref[...]

    m2 = jnp.mean(x1, axis=-1, keepdims=True)
    ms2 = jnp.mean(x1 * x1, axis=-1, keepdims=True)
    r2 = jax.lax.rsqrt(ms2 - m2 * m2 + 1e-5)
    xn2 = (x1 - m2) * r2 * ln2g_ref[...] + ln2b_ref[...]

    sT = jax.lax.dot_general(
        wrt_ref[...], xn2, (((1,), (1,)), ((), ())),
        preferred_element_type=f32) + brt_ref[...]
    logitsT = sT[:NE]
    nlogT = sT[NE:2 * NE]
    skipT = sT[2 * NE:2 * NE + 1]
    noisyT = logitsT + epsT_ref[...] * jax.nn.softplus(nlogT)

    idxs = jax.lax.broadcasted_iota(jnp.int32, (NE, P), 0)
    v1 = jnp.max(noisyT, axis=0, keepdims=True)
    i1 = jnp.min(jnp.where(noisyT == v1, idxs, NE), axis=0, keepdims=True)
    n2 = jnp.where(idxs == i1, -jnp.inf, noisyT)
    v2 = jnp.max(n2, axis=0, keepdims=True)
    i2 = jnp.min(jnp.where(n2 == v2, idxs, NE), axis=0, keepdims=True)
    sel = (idxs == i1) | (idxs == i2)
    prT = jnp.where(sel, jnp.exp(noisyT - v1), 0.0)
    prT = prT / jnp.sum(prT, axis=0, keepdims=True)

    skip_col = jnp.transpose(skipT)
    pr_col = jnp.transpose(prT)

    f8 = jnp.float8_e4m3fn
    xn2q = xn2.astype(f8)
    acc = jnp.zeros((P, C), f32)
    for e in range(NE):
        h = jnp.dot(xn2q, we1_ref[e], preferred_element_type=f32)
        hb = jnp.maximum(h + be1_ref[e], 0).astype(f8)
        ye = jnp.dot(hb, we2_ref[e], preferred_element_type=f32)
        acc = acc + ye * pr_col[:, e:e + 1]
    acc = acc * (1.0 / 256.0) \
        + jnp.dot(pr_col, be2_ref[...], preferred_element_type=f32)

    out = jnp.where(skip_col > 0.0, x1, x1 + acc)
    out_ref[...] = out.reshape(BB, T, C)


def kernel(x, Wq, Wk, Wv, Wp, bp, ln1_g, ln1_b, ln2_g, ln2_b,
           Wer, ber, Wn, bn, Ws, bs, We1, be1, We2, be2):
    bf = jnp.bfloat16
    Wqf = (Wq.transpose(1, 0, 2).reshape(C, NH * HD) * C ** -0.5).astype(bf)
    Wkf = Wk.transpose(1, 0, 2).reshape(C, NH * HD).astype(bf)
    Wvf = Wv.transpose(1, 0, 2).reshape(C, NH * HD).astype(bf)
    Wpb = Wp.astype(bf)
    f8 = jnp.float8_e4m3fn
    We1b = (We1 * 16.0).astype(f8)
    We2b = (We2 * 16.0).astype(f8)
    be1b = be1 * 16.0
    WrT = jnp.concatenate([Wer, Wn, Ws], axis=1).T
    brT = jnp.concatenate([ber, bn, bs]).reshape(2 * NE + 1, 1)
    eps = jax.random.normal(jax.random.key(42), (B, T, NE), jnp.float32)
    epsT = eps.reshape(B * T, NE).T

    row = lambda a: a.reshape(1, -1)
    full = lambda arr: pl.BlockSpec(arr.shape, lambda i: (0,) * arr.ndim)

    weights = (Wqf, Wkf, Wvf, Wpb, row(bp), row(ln1_g), row(ln1_b),
               row(ln2_g), row(ln2_b), WrT, brT, We1b, be1b, We2b, be2)

    out = pl.pallas_call(
        _fused_kernel,
        grid=(B // BB,),
        in_specs=[pl.BlockSpec((BB, T, C), lambda i: (i, 0, 0)),
                  pl.BlockSpec((NE, P), lambda i: (0, i))]
                 + [full(w) for w in weights],
        out_specs=pl.BlockSpec((BB, T, C), lambda i: (i, 0, 0)),
        out_shape=jax.ShapeDtypeStruct((B, T, C), jnp.float32),
    )(x, epsT, *weights)
    return out
